# baseline (speedup 1.0000x reference)
import functools

import jax
import jax.numpy as jnp
from jax import lax
from jax.experimental import pallas as pl
from jax.experimental.pallas import tpu as pltpu
from jax.experimental.pallas import tpu_sc as plsc

BATCH = 4
DIMZ = DIMY = DIMX = 128
WIDTH = 320
HEIGHT = 240
DEPTH_MIN = 0.4
RAY_INC = 0.05
NSTEPS = 72
NPIX = HEIGHT * WIDTH
ZGROUPS = DIMZ // 32
NWORDS = ZGROUPS * DIMY * DIMX

NWORKERS = 32
CHUNKS = NWORKERS // BATCH
CHUNK_PIX = NPIX // CHUNKS
VREGS = CHUNK_PIX // 16

MAGIC = 12582912.0


def _bf16q(x):
    q = lax.bitcast_convert_type(x, jnp.int32)
    bias = 0x7FFF + ((q >> 16) & 1)
    q2 = (q + bias) & jnp.int32(~0xFFFF)
    return lax.bitcast_convert_type(q2, jnp.float32)


def _pack_kernel(occ_ref, out_ref):
    x = occ_ref[0]
    lo = x[0]
    for j in range(1, 16):
        lo = lo + x[j] * float(2 ** j)
    hi = x[16]
    for j in range(1, 16):
        hi = hi + x[16 + j] * float(2 ** j)
    out_ref[0] = lo.astype(jnp.int32) | (hi.astype(jnp.int32) << 16)


def _pack_volume(occ3d):
    occ = occ3d.reshape(BATCH * ZGROUPS, 32, DIMY, DIMX)
    return pl.pallas_call(
        _pack_kernel,
        grid=(BATCH * ZGROUPS,),
        in_specs=[pl.BlockSpec((1, 32, DIMY, DIMX), lambda g: (g, 0, 0, 0))],
        out_specs=pl.BlockSpec((1, DIMY, DIMX), lambda g: (g, 0, 0)),
        out_shape=jax.ShapeDtypeStruct((BATCH * ZGROUPS, DIMY, DIMX), jnp.int32),
    )(occ)


def _raycast_body(packed_hbm, ugrid_hbm, vgrid_hbm, params_hbm, out_hbm,
                  packed_v, u_v, v_v, par_v, out_v, acc_v, sem):
    c = lax.axis_index("c")
    s = lax.axis_index("s")
    wid = s * 2 + c
    b = wid & 3
    chunk = wid >> 2
    pix0 = chunk * CHUNK_PIX

    pltpu.sync_copy(packed_hbm.at[b], packed_v)
    pltpu.sync_copy(ugrid_hbm.at[pl.ds(pix0, CHUNK_PIX)], u_v)
    pltpu.sync_copy(vgrid_hbm.at[pl.ds(pix0, CHUNK_PIX)], v_v)
    pltpu.sync_copy(params_hbm.at[b], par_v)

    m00 = _bf16q(par_v[0, :])
    m01 = _bf16q(par_v[1, :])
    m02 = _bf16q(par_v[2, :])
    m10 = _bf16q(par_v[3, :])
    m11 = _bf16q(par_v[4, :])
    m12 = _bf16q(par_v[5, :])
    m20 = _bf16q(par_v[6, :])
    m21 = _bf16q(par_v[7, :])
    m22 = _bf16q(par_v[8, :])
    tx = _bf16q(par_v[9, :])
    ty = _bf16q(par_v[10, :])
    tz = _bf16q(par_v[11, :])
    fx = par_v[12, :]
    fy = par_v[13, :]
    cx = par_v[14, :]
    cy = par_v[15, :]

    def prep_iter(i, carry):
        off = i * 16
        u = u_v[pl.ds(off, 16)]
        v = v_v[pl.ds(off, 16)]
        u_v[pl.ds(off, 16)] = (u - cx) / fx
        v_v[pl.ds(off, 16)] = (v - cy) / fy
        acc_v[pl.ds(off, 16)] = jnp.zeros((16,), jnp.int32)
        return carry

    lax.fori_loop(0, VREGS, prep_iter, 0)

    def step(k, carry):
        d = DEPTH_MIN + k.astype(jnp.float32) * RAY_INC
        dv = jnp.full((16,), 0.0, jnp.float32) + d
        dq = _bf16q(dv)
        zxs = m02 * dq
        zys = m12 * dq
        szs = m22 * dq + tz

        @plsc.parallel_loop(0, CHUNK_PIX, step=16, unroll=1)
        def pixel_iter(off):
            a0 = u_v[pl.ds(off, 16)]
            a1 = v_v[pl.ds(off, 16)]
            xcq = _bf16q(a0 * dv)
            ycq = _bf16q(a1 * dv)
            x = (m00 * xcq + tx) + (m01 * ycq + zxs)
            y = (m11 * ycq + ty) + (m10 * xcq + zys)
            z = szs + (m20 * xcq + m21 * ycq)
            hic = jnp.maximum(x, jnp.maximum(y, z))
            loc = jnp.minimum(x, jnp.minimum(y, z))
            valid = (loc >= -0.5) & (hic < 127.5)
            xr = (x + MAGIC) - MAGIC
            yr = (y + MAGIC) - MAGIC
            zr = (z + MAGIC) - MAGIC
            xi = xr.astype(jnp.int32)
            yi = yr.astype(jnp.int32)
            zi = zr.astype(jnp.int32)
            widx = (((zi >> 5) << 14) | (yi << 7) | xi) & 0xFFFF
            w = plsc.load_gather(packed_v, [widx])
            hit = (w >> (zi & 31)) & 1
            acc_v[pl.ds(off, 16)] = acc_v[pl.ds(off, 16)] | jnp.where(valid, hit, 0)

        return carry

    lax.fori_loop(0, NSTEPS, step, 0)

    def emit_iter(i, carry):
        off = i * 16
        out_v[pl.ds(off, 16)] = acc_v[pl.ds(off, 16)].astype(jnp.float32)
        return carry

    lax.fori_loop(0, VREGS, emit_iter, 0)
    pltpu.sync_copy(out_v, out_hbm.at[b, pl.ds(pix0, CHUNK_PIX)])


def _raycast(packed, ugrid, vgrid, params):
    mesh = plsc.VectorSubcoreMesh(core_axis_name="c", subcore_axis_name="s")
    fn = pl.kernel(
        _raycast_body,
        out_type=jax.ShapeDtypeStruct((BATCH, NPIX), jnp.float32),
        mesh=mesh,
        compiler_params=pltpu.CompilerParams(needs_layout_passes=False),
        scratch_types=[
            pltpu.VMEM((NWORDS,), jnp.int32),
            pltpu.VMEM((CHUNK_PIX,), jnp.float32),
            pltpu.VMEM((CHUNK_PIX,), jnp.float32),
            pltpu.VMEM((16, 16), jnp.float32),
            pltpu.VMEM((CHUNK_PIX,), jnp.float32),
            pltpu.VMEM((CHUNK_PIX,), jnp.int32),
            pltpu.SemaphoreType.DMA,
        ],
    )
    return fn(packed, ugrid, vgrid, params)


def kernel(occ3d, view_matrix, intrinsic_params):
    packed = _pack_volume(occ3d).reshape(BATCH, NWORDS)

    u = jnp.tile(jnp.arange(WIDTH, dtype=jnp.float32), (HEIGHT,))
    v = jnp.repeat(jnp.arange(HEIGHT, dtype=jnp.float32), WIDTH)

    p = jnp.stack([
        view_matrix[:, 0, 0], view_matrix[:, 0, 1], view_matrix[:, 0, 2],
        view_matrix[:, 1, 0], view_matrix[:, 1, 1], view_matrix[:, 1, 2],
        view_matrix[:, 2, 0], view_matrix[:, 2, 1], view_matrix[:, 2, 2],
        view_matrix[:, 0, 3], view_matrix[:, 1, 3], view_matrix[:, 2, 3],
        intrinsic_params[:, 0], intrinsic_params[:, 1],
        intrinsic_params[:, 2], intrinsic_params[:, 3],
    ], axis=1)
    params = jnp.tile(p[:, :, None], (1, 1, 16))

    out = _raycast(packed, u, v, params)
    return out.reshape(BATCH, 1, HEIGHT, WIDTH)


# speedup vs baseline: 1.0847x; 1.0847x over previous
import functools

import jax
import jax.numpy as jnp
from jax import lax
from jax.experimental import pallas as pl
from jax.experimental.pallas import tpu as pltpu
from jax.experimental.pallas import tpu_sc as plsc

BATCH = 4
DIMZ = DIMY = DIMX = 128
WIDTH = 320
HEIGHT = 240
DEPTH_MIN = 0.4
RAY_INC = 0.05
NSTEPS = 72
NPIX = HEIGHT * WIDTH
ZGROUPS = DIMZ // 32
NWORDS = ZGROUPS * DIMY * DIMX

NWORKERS = 32
CHUNKS = NWORKERS // BATCH
CHUNK_PIX = NPIX // CHUNKS
VREGS = CHUNK_PIX // 16

MAGIC = 12582912.0


def _bf16q(x):
    q = lax.bitcast_convert_type(x, jnp.int32)
    bias = 0x7FFF + ((q >> 16) & 1)
    q2 = (q + bias) & jnp.int32(~0xFFFF)
    return lax.bitcast_convert_type(q2, jnp.float32)


def _pack_kernel(occ_ref, out_ref):
    x = occ_ref[0]
    lo = x[0]
    for j in range(1, 16):
        lo = lo + x[j] * float(2 ** j)
    hi = x[16]
    for j in range(1, 16):
        hi = hi + x[16 + j] * float(2 ** j)
    out_ref[0] = lo.astype(jnp.int32) | (hi.astype(jnp.int32) << 16)


def _pack_volume(occ3d):
    occ = occ3d.reshape(BATCH * ZGROUPS, 32, DIMY, DIMX)
    return pl.pallas_call(
        _pack_kernel,
        grid=(BATCH * ZGROUPS,),
        in_specs=[pl.BlockSpec((1, 32, DIMY, DIMX), lambda g: (g, 0, 0, 0))],
        out_specs=pl.BlockSpec((1, DIMY, DIMX), lambda g: (g, 0, 0)),
        out_shape=jax.ShapeDtypeStruct((BATCH * ZGROUPS, DIMY, DIMX), jnp.int32),
    )(occ)


def _raycast_body(packed_hbm, ugrid_hbm, vgrid_hbm, params_hbm, out_hbm,
                  packed_v, u_v, v_v, par_v, out_v, acc_v, sem):
    c = lax.axis_index("c")
    s = lax.axis_index("s")
    wid = s * 2 + c
    b = wid & 3
    chunk = wid >> 2
    pix0 = chunk * CHUNK_PIX

    pltpu.sync_copy(packed_hbm.at[b], packed_v)
    pltpu.sync_copy(ugrid_hbm.at[pl.ds(pix0, CHUNK_PIX)], u_v)
    pltpu.sync_copy(vgrid_hbm.at[pl.ds(pix0, CHUNK_PIX)], v_v)
    pltpu.sync_copy(params_hbm.at[b], par_v)

    m00 = _bf16q(par_v[0, :])
    m01 = _bf16q(par_v[1, :])
    m02 = _bf16q(par_v[2, :])
    m10 = _bf16q(par_v[3, :])
    m11 = _bf16q(par_v[4, :])
    m12 = _bf16q(par_v[5, :])
    m20 = _bf16q(par_v[6, :])
    m21 = _bf16q(par_v[7, :])
    m22 = _bf16q(par_v[8, :])
    tx = _bf16q(par_v[9, :])
    ty = _bf16q(par_v[10, :])
    tz = _bf16q(par_v[11, :])
    fx = par_v[12, :]
    fy = par_v[13, :]
    cx = par_v[14, :]
    cy = par_v[15, :]

    def prep_iter(i, carry):
        off = i * 16
        u = u_v[pl.ds(off, 16)]
        v = v_v[pl.ds(off, 16)]
        u_v[pl.ds(off, 16)] = (u - cx) / fx
        v_v[pl.ds(off, 16)] = (v - cy) / fy
        acc_v[pl.ds(off, 16)] = jnp.zeros((16,), jnp.int32)
        return carry

    lax.fori_loop(0, VREGS, prep_iter, 0)

    def step(k, carry):
        d = DEPTH_MIN + k.astype(jnp.float32) * RAY_INC
        dv = jnp.full((16,), 0.0, jnp.float32) + d
        dq = _bf16q(dv)
        zxs = m02 * dq
        zys = m12 * dq
        szs = m22 * dq + tz

        def pixel_iter(i, c2):
            off = i * 16
            a0 = u_v[pl.ds(off, 16)]
            a1 = v_v[pl.ds(off, 16)]
            xcq = _bf16q(a0 * dv)
            ycq = _bf16q(a1 * dv)
            x = (m00 * xcq + tx) + (m01 * ycq + zxs)
            y = (m11 * ycq + ty) + (m10 * xcq + zys)
            z = szs + (m20 * xcq + m21 * ycq)
            hic = jnp.maximum(x, jnp.maximum(y, z))
            loc = jnp.minimum(x, jnp.minimum(y, z))
            valid = (loc >= -0.5) & (hic < 127.5)
            xr = (x + MAGIC) - MAGIC
            yr = (y + MAGIC) - MAGIC
            zr = (z + MAGIC) - MAGIC
            xi = xr.astype(jnp.int32)
            yi = yr.astype(jnp.int32)
            zi = zr.astype(jnp.int32)
            widx = (((zi >> 5) << 14) | (yi << 7) | xi) & 0xFFFF
            w = plsc.load_gather(packed_v, [widx])
            hit = (w >> (zi & 31)) & 1
            acc_v[pl.ds(off, 16)] = acc_v[pl.ds(off, 16)] | jnp.where(valid, hit, 0)
            return c2

        lax.fori_loop(0, VREGS, pixel_iter, 0)
        return carry

    lax.fori_loop(0, NSTEPS, step, 0)

    def emit_iter(i, carry):
        off = i * 16
        out_v[pl.ds(off, 16)] = acc_v[pl.ds(off, 16)].astype(jnp.float32)
        return carry

    lax.fori_loop(0, VREGS, emit_iter, 0)
    pltpu.sync_copy(out_v, out_hbm.at[b, pl.ds(pix0, CHUNK_PIX)])


def _raycast(packed, ugrid, vgrid, params):
    mesh = plsc.VectorSubcoreMesh(core_axis_name="c", subcore_axis_name="s")
    fn = pl.kernel(
        _raycast_body,
        out_type=jax.ShapeDtypeStruct((BATCH, NPIX), jnp.float32),
        mesh=mesh,
        compiler_params=pltpu.CompilerParams(needs_layout_passes=False),
        scratch_types=[
            pltpu.VMEM((NWORDS,), jnp.int32),
            pltpu.VMEM((CHUNK_PIX,), jnp.float32),
            pltpu.VMEM((CHUNK_PIX,), jnp.float32),
            pltpu.VMEM((16, 16), jnp.float32),
            pltpu.VMEM((CHUNK_PIX,), jnp.float32),
            pltpu.VMEM((CHUNK_PIX,), jnp.int32),
            pltpu.SemaphoreType.DMA,
        ],
    )
    return fn(packed, ugrid, vgrid, params)


def kernel(occ3d, view_matrix, intrinsic_params):
    packed = _pack_volume(occ3d).reshape(BATCH, NWORDS)

    u = jnp.tile(jnp.arange(WIDTH, dtype=jnp.float32), (HEIGHT,))
    v = jnp.repeat(jnp.arange(HEIGHT, dtype=jnp.float32), WIDTH)

    p = jnp.stack([
        view_matrix[:, 0, 0], view_matrix[:, 0, 1], view_matrix[:, 0, 2],
        view_matrix[:, 1, 0], view_matrix[:, 1, 1], view_matrix[:, 1, 2],
        view_matrix[:, 2, 0], view_matrix[:, 2, 1], view_matrix[:, 2, 2],
        view_matrix[:, 0, 3], view_matrix[:, 1, 3], view_matrix[:, 2, 3],
        intrinsic_params[:, 0], intrinsic_params[:, 1],
        intrinsic_params[:, 2], intrinsic_params[:, 3],
    ], axis=1)
    params = jnp.tile(p[:, :, None], (1, 1, 16))

    out = _raycast(packed, u, v, params)
    return out.reshape(BATCH, 1, HEIGHT, WIDTH)
